# baseline (speedup 1.0000x reference)
import math

import jax
import jax.numpy as jnp
from jax.experimental import pallas as pl
from jax.experimental.pallas import tpu as pltpu

_HPB = 3


def _pair_pipeline(q, k, v):
    S, W2 = q.shape
    D = W2 // 2
    L = S.bit_length() - 1
    NS = 2 * ((L + 15) // 16) * 16
    scale = 1.0 / math.sqrt(D)
    f32 = jnp.float32
    dnums = (((1,), (0,)), ((), ()))

    def mm(a, b):
        return jax.lax.dot_general(a, b, dnums, preferred_element_type=f32)

    lane_r = jax.lax.broadcasted_iota(jnp.int32, (W2, 2), 0) // D
    col2 = jax.lax.broadcasted_iota(jnp.int32, (W2, 2), 1)
    wred = (lane_r == col2).astype(f32)

    lane_rs = jax.lax.broadcasted_iota(jnp.int32, (W2, NS), 0) // D
    col_s = jax.lax.broadcasted_iota(jnp.int32, (W2, NS), 1)

    def wred_at(l):
        return (col_s == 2 * l + lane_rs).astype(f32)

    row2 = jax.lax.broadcasted_iota(jnp.int32, (2, NS), 0)
    colsn = jax.lax.broadcasted_iota(jnp.int32, (2, NS), 1)
    tsel = (row2 == (colsn % 2)).astype(f32)

    rowsn = jax.lax.broadcasted_iota(jnp.int32, (NS, 2), 0)
    coln2 = jax.lax.broadcasted_iota(jnp.int32, (NS, 2), 1)
    gsum = (coln2 == (rowsn % 2)).astype(f32)
    gsum = gsum * (rowsn < 2 * L).astype(f32)

    rowb = jax.lax.broadcasted_iota(jnp.int32, (2, W2), 0)
    laneb = jax.lax.broadcasted_iota(jnp.int32, (2, W2), 1) // D
    bful = (rowb == laneb).astype(f32)

    rowbs = jax.lax.broadcasted_iota(jnp.int32, (NS, W2), 0)
    lanebs = jax.lax.broadcasted_iota(jnp.int32, (NS, W2), 1) // D

    def bsel_at(l):
        return (rowbs == 2 * l + lanebs).astype(f32)

    row = jax.lax.broadcasted_iota(jnp.int32, (S, 1), 0)
    lvlL = jax.lax.broadcasted_iota(jnp.int32, (1, NS), 1) // 2
    maskf = ((row >> lvlL) & 1).astype(f32)

    m_raw = mm(q * k, wred)

    scores = jnp.zeros((S, NS), f32)
    kl, vl = k, v
    ev_vs = []
    for l in range(L):
        n = S >> l
        kr = kl.reshape(n // 2, 2, W2)
        vr = vl.reshape(n // 2, 2, W2)
        kc0 = kr[:, 0, :]
        kc1 = kr[:, 1, :]
        vc0 = vr[:, 0, :]
        vc1 = vr[:, 1, :]
        ev_vs.append(vc0)

        rep = 1 << (l + 1)
        nbr_k = jnp.broadcast_to(kc0[:, None, :], (n // 2, rep, W2)).reshape(S, W2)
        scores = scores + mm(q * nbr_k, wred_at(l))

        if l + 1 < L:
            kp = 0.5 * (kc0 + kc1)
            s0 = mm(kp * kc0, wred) * scale
            s1 = mm(kp * kc1, wred) * scale
            ss = 0.5 * (s0 + s1)
            mx = jnp.maximum(jnp.maximum(ss, s0), s1)
            es = jnp.exp(ss - mx)
            e0 = jnp.exp(s0 - mx)
            e1 = jnp.exp(s1 - mx)
            rden = 1.0 / (es + e0 + e1)
            c0 = (0.5 * es + e0) * rden
            c1 = (0.5 * es + e1) * rden
            vl = mm(c0, bful) * vc0 + mm(c1, bful) * vc1
            kl = kp

    E = jnp.exp(scale * (scores - mm(m_raw, tsel))) * maskf
    d = 1.0 + mm(E, gsum)

    acc = v
    for l in range(L):
        n2 = (S >> l) // 2
        rep = 1 << (l + 1)
        nbr_v = jnp.broadcast_to(
            ev_vs[l][:, None, :], (n2, rep, W2)).reshape(S, W2)
        acc = acc + mm(E, bsel_at(l)) * nbr_v

    return acc * mm(1.0 / d, bful)


def _attn_kernel(q_ref, k_ref, v_ref, o_ref):
    W2 = 128
    for hp in range(_HPB):
        sl = slice(hp * W2, (hp + 1) * W2)
        o_ref[0, :, sl] = _pair_pipeline(
            q_ref[0, :, sl], k_ref[0, :, sl], v_ref[0, :, sl])


@jax.jit
def kernel(q, k, v):
    B, S, H, D = q.shape
    Hp = H // 2
    G = Hp // _HPB
    lanes = _HPB * 2 * D

    def flat(x):
        return x.reshape(B, S, H * D)

    spec = pl.BlockSpec((1, S, lanes), lambda b, g: (b, 0, g))
    out = pl.pallas_call(
        _attn_kernel,
        grid=(B, G),
        in_specs=[spec, spec, spec],
        out_specs=spec,
        out_shape=jax.ShapeDtypeStruct((B, S, H * D), q.dtype),
        compiler_params=pltpu.CompilerParams(
            dimension_semantics=("parallel", "parallel")),
    )(flat(q), flat(k), flat(v))
    return out.reshape(B, S, H, D)


# speedup vs baseline: 1.0568x; 1.0568x over previous
import math

import jax
import jax.numpy as jnp
from jax.experimental import pallas as pl
from jax.experimental.pallas import tpu as pltpu

_HPB = 2


def _pair_pipeline(q, k, v):
    S, W2 = q.shape
    D = W2 // 2
    L = S.bit_length() - 1
    NS = 2 * ((L + 15) // 16) * 16
    scale = 1.0 / math.sqrt(D)
    f32 = jnp.float32
    dnums = (((1,), (0,)), ((), ()))

    def mm(a, b):
        return jax.lax.dot_general(a, b, dnums, preferred_element_type=f32)

    lane_r = jax.lax.broadcasted_iota(jnp.int32, (W2, 2), 0) // D
    col2 = jax.lax.broadcasted_iota(jnp.int32, (W2, 2), 1)
    wred = (lane_r == col2).astype(f32)

    lane_rs = jax.lax.broadcasted_iota(jnp.int32, (W2, NS), 0) // D
    col_s = jax.lax.broadcasted_iota(jnp.int32, (W2, NS), 1)

    def wred_at(l):
        return (col_s == 2 * l + lane_rs).astype(f32)

    row2 = jax.lax.broadcasted_iota(jnp.int32, (2, NS), 0)
    colsn = jax.lax.broadcasted_iota(jnp.int32, (2, NS), 1)
    tsel = (row2 == (colsn % 2)).astype(f32)

    rowsn = jax.lax.broadcasted_iota(jnp.int32, (NS, 2), 0)
    coln2 = jax.lax.broadcasted_iota(jnp.int32, (NS, 2), 1)
    gsum = (coln2 == (rowsn % 2)).astype(f32)
    gsum = gsum * (rowsn < 2 * L).astype(f32)

    rowb = jax.lax.broadcasted_iota(jnp.int32, (2, W2), 0)
    laneb = jax.lax.broadcasted_iota(jnp.int32, (2, W2), 1) // D
    bful = (rowb == laneb).astype(f32)

    rowbs = jax.lax.broadcasted_iota(jnp.int32, (NS, W2), 0)
    lanebs = jax.lax.broadcasted_iota(jnp.int32, (NS, W2), 1) // D

    def bsel_at(l):
        return (rowbs == 2 * l + lanebs).astype(f32)

    row = jax.lax.broadcasted_iota(jnp.int32, (S, 1), 0)
    lvlL = jax.lax.broadcasted_iota(jnp.int32, (1, NS), 1) // 2
    maskf = ((row >> lvlL) & 1).astype(f32)

    m_raw = mm(q * k, wred)

    scores = jnp.zeros((S, NS), f32)
    kl, vl = k, v
    ev_vs = []
    for l in range(L):
        n = S >> l
        kr = kl.reshape(n // 2, 2, W2)
        vr = vl.reshape(n // 2, 2, W2)
        kc0 = kr[:, 0, :]
        kc1 = kr[:, 1, :]
        vc0 = vr[:, 0, :]
        vc1 = vr[:, 1, :]
        ev_vs.append(vc0)

        rep = 1 << (l + 1)
        if rep >= 8:
            prod = (q.reshape(n // 2, rep, W2) * kc0[:, None, :]).reshape(S, W2)
        else:
            nbr_k = jnp.broadcast_to(
                kc0[:, None, :], (n // 2, rep, W2)).reshape(S, W2)
            prod = q * nbr_k
        scores = scores + mm(prod, wred_at(l))

        if l + 1 < L:
            n2 = n // 2
            kp = 0.5 * (kc0 + kc1)
            s01 = mm(jnp.concatenate([kp * kc0, kp * kc1], axis=0),
                     wred) * scale
            s0 = s01[:n2]
            s1 = s01[n2:]
            ss = 0.5 * (s0 + s1)
            mx = jnp.maximum(jnp.maximum(ss, s0), s1)
            es = jnp.exp(ss - mx)
            e0 = jnp.exp(s0 - mx)
            e1 = jnp.exp(s1 - mx)
            rden = 1.0 / (es + e0 + e1)
            c0 = (0.5 * es + e0) * rden
            c1 = (0.5 * es + e1) * rden
            cb = mm(jnp.concatenate([c0, c1], axis=0), bful)
            vl = cb[:n2] * vc0 + cb[n2:] * vc1
            kl = kp

    E = jnp.exp(scale * (scores - mm(m_raw, tsel))) * maskf
    d = 1.0 + mm(E, gsum)

    RW = 256
    lhi = [l for l in range(L) if (1 << (l + 1)) >= RW]
    acc = v
    for l in range(L):
        if l in lhi:
            continue
        n2 = (S >> l) // 2
        rep = 1 << (l + 1)
        w = mm(E, bsel_at(l))
        if rep >= 8:
            acc = acc + (w.reshape(n2, rep, W2)
                         * ev_vs[l][:, None, :]).reshape(S, W2)
        else:
            nbr_v = jnp.broadcast_to(
                ev_vs[l][:, None, :], (n2, rep, W2)).reshape(S, W2)
            acc = acc + w * nbr_v

    if lhi:
        hmask0 = (jax.lax.broadcasted_iota(jnp.int32, (1, W2), 1)
                  // D == 0).astype(f32)
        hmask1 = 1.0 - hmask0
        ztop = jnp.zeros((2 * lhi[0], W2), f32)
        ztail = jnp.zeros((NS - 2 * (lhi[-1] + 1), W2), f32)
        lg = RW.bit_length() - 1
        parts = []
        for w in range(S // RW):
            rows = [ztop]
            for l in lhi:
                node = (w * RW) >> (l + 1)
                vrow = ev_vs[l][node:node + 1, :]
                rows.append(vrow * hmask0)
                rows.append(vrow * hmask1)
            rows.append(ztail)
            vsel = jnp.concatenate(rows, axis=0)
            parts.append(mm(E[w * RW:(w + 1) * RW, :], vsel))
        acc = acc + jnp.concatenate(parts, axis=0)

    return acc * mm(1.0 / d, bful)


def _attn_kernel(q_ref, k_ref, v_ref, o_ref):
    W2 = 128
    for hp in range(_HPB):
        sl = slice(hp * W2, (hp + 1) * W2)
        o_ref[0, :, sl] = _pair_pipeline(
            q_ref[0, :, sl], k_ref[0, :, sl], v_ref[0, :, sl])


@jax.jit
def kernel(q, k, v):
    B, S, H, D = q.shape
    Hp = H // 2
    G = Hp // _HPB
    lanes = _HPB * 2 * D

    def flat(x):
        return x.reshape(B, S, H * D)

    spec = pl.BlockSpec((1, S, lanes), lambda b, g: (b, 0, g))
    out = pl.pallas_call(
        _attn_kernel,
        grid=(B, G),
        in_specs=[spec, spec, spec],
        out_specs=spec,
        out_shape=jax.ShapeDtypeStruct((B, S, H * D), q.dtype),
        compiler_params=pltpu.CompilerParams(
            dimension_semantics=("parallel", "parallel")),
    )(flat(q), flat(k), flat(v))
    return out.reshape(B, S, H, D)
